# baseline (speedup 1.0000x reference)
import functools

import jax
import jax.numpy as jnp
from jax import lax
from jax.experimental import pallas as pl
from jax.experimental.pallas import tpu as pltpu
from jax.experimental.pallas import tpu_sc as plsc

B = 16384
D = 64
TAG = 128
NW = 32
WV = 256
WVS = 8
WVP = WV
BOUT = B + NW
LANES = 16


def _splat(x, dtype=jnp.int32):
    return jnp.full((LANES,), x, dtype)


def _make_stream_gather(V):
    n_full = V // WV
    tail_w = V - n_full * WV
    tail_off = n_full * WV

    def body(idx_hbm, tableT_hbm, out_hbm,
             ibuf, hbuf, winbuf, tailbuf, curl,
             staging, posline, semw):
        wid = lax.axis_index("s") * 2 + lax.axis_index("c")
        kmax = (n_full - 1 - wid) // NW + 1
        iota = lax.iota(jnp.int32, LANES)

        def zrow(i, c):
            for j in range(4):
                plsc.store_scatter(staging,
                                   [_splat(i), D + 16 * j + iota],
                                   jnp.zeros((LANES,), jnp.float32))
            return c
        lax.fori_loop(0, 128, zrow, 0)

        for j in range(8):
            plsc.store_scatter(posline, [_splat(0), 16 * j + iota],
                               _splat(B + wid))

        IC = 2048

        def chunk(q, off):
            pltpu.sync_copy(idx_hbm.at[pl.ds(q * IC, IC)], ibuf)

            def scan(g, off):
                v = ibuf[pl.ds(g * LANES, LANES)]
                pos = q * IC + g * LANES + iota
                m = ((v >> WVS) & (NW - 1)) == wid
                packed = ((v >> (WVS + 5)) << 23) | ((v & (WV - 1)) << 14) | pos
                plsc.store_compressed(hbuf.at[pl.ds(off, LANES)], packed,
                                      mask=m)
                return off + plsc.all_reduce_population_count(m)[0]

            return lax.fori_loop(0, IC // LANES, scan, off)

        n = lax.fori_loop(0, B // IC, chunk, 0)
        hbuf[pl.ds(n, LANES)] = _splat(jnp.int32(2147400000))

        nvec = (n + LANES - 1) // LANES

        def process_hits(k_id, slot_idx, src_ref, f):
            def hit_vec(t, f):
                hv = hbuf[pl.ds(t * LANES, LANES)]
                m = (hv >> 23) == k_id
                plsc.store_compressed(curl.at[pl.ds(0, LANES)], hv, mask=m)
                c = plsc.all_reduce_population_count(m)[0]

                def item(j, f):
                    w = curl[pl.ds(j, LANES)][0]
                    l = (w >> 14) & 511
                    p = w & (B - 1)
                    fm = lax.rem(f, 128)
                    for i in range(4):
                        dvec = 16 * i + iota
                        if src_ref is winbuf:
                            vals = plsc.load_gather(
                                winbuf, [_splat(slot_idx), dvec, _splat(l)])
                        else:
                            vals = plsc.load_gather(src_ref, [dvec, _splat(l)])
                        plsc.store_scatter(staging, [_splat(fm), dvec], vals)
                    plsc.store_scatter(posline, [_splat(0), _splat(fm)],
                                       _splat(p), mask=iota == 0)
                    f = f + 1

                    @pl.when(lax.rem(f, 128) == 0)
                    def _flush():
                        pltpu.sync_copy(staging, out_hbm.at[posline.at[0]])
                    return f

                return lax.fori_loop(0, c, item, f)

            return lax.fori_loop(0, nvec, hit_vec, f)

        def win_src(k):
            off = pl.multiple_of((wid + k * NW) * WV, WV)
            return tableT_hbm.at[:, pl.ds(off, WV)]

        pltpu.async_copy(win_src(0), winbuf.at[0], semw)

        def wloop(k, f):
            @pl.when(k + 1 < kmax)
            def _():
                pltpu.async_copy(win_src(k + 1), winbuf.at[lax.rem(k + 1, 2)],
                                 semw)
            pltpu.make_async_copy(win_src(k), winbuf.at[lax.rem(k, 2)],
                                  semw).wait()
            return process_hits(k, lax.rem(k, 2), winbuf, f)

        f = lax.fori_loop(0, kmax, wloop, 0)

        if tail_w:
            pltpu.sync_copy(tableT_hbm.at[:, pl.ds(tail_off, tail_w)], tailbuf)
            f = process_hits(n_full // NW, 0, tailbuf, f)

        @pl.when(lax.rem(f, 128) != 0)
        def _():
            pltpu.sync_copy(staging, out_hbm.at[posline.at[0]])

    return functools.partial(
        pl.kernel,
        mesh=plsc.VectorSubcoreMesh(core_axis_name="c", subcore_axis_name="s"),
        out_type=jax.ShapeDtypeStruct((BOUT, TAG), jnp.float32),
        scratch_types=[
            pltpu.VMEM((2048,), jnp.int32),
            pltpu.VMEM((B + LANES,), jnp.int32),
            pltpu.VMEM((2, D, WVP), jnp.float32),
            pltpu.VMEM((D, max(tail_w, 8)), jnp.float32),
            pltpu.VMEM((2 * LANES,), jnp.int32),
            pltpu.VMEM((128, TAG), jnp.float32),
            pltpu.VMEM((1, TAG), jnp.int32),
            pltpu.SemaphoreType.DMA,
        ],
        compiler_params=pltpu.CompilerParams(use_tc_tiling_on_sc=True, needs_layout_passes=False),
    )(body)


_gather_user = _make_stream_gather(1000000)
_gather_artist = _make_stream_gather(100000)


BB = 2048


def _mlp_body(u2_ref, a2_ref, t_ref, wtag_ref, w1u_ref, w1a_ref, w1t_ref,
              b1_ref, w2_ref, b2_ref, w3_ref, b3_ref, out_ref):
    f32 = jnp.float32
    u = u2_ref[:, :D]
    a = a2_ref[:, :D]
    t = jnp.maximum(
        jnp.dot(t_ref[...], wtag_ref[...], preferred_element_type=f32), 0.0)
    h = jnp.dot(u, w1u_ref[...], preferred_element_type=f32)
    h = h + jnp.dot(a, w1a_ref[...], preferred_element_type=f32)
    h = h + jnp.dot(t, w1t_ref[...], preferred_element_type=f32)
    h = jnp.maximum(h + b1_ref[...], 0.0)
    h = jnp.maximum(
        jnp.dot(h, w2_ref[...], preferred_element_type=f32) + b2_ref[...], 0.0)
    logit = jnp.dot(h, w3_ref[...], preferred_element_type=f32) + b3_ref[...]
    out_ref[...] = jax.nn.sigmoid(logit)


def _full(shape):
    return pl.BlockSpec(shape, lambda i: (0, 0))


_mlp = pl.pallas_call(
    _mlp_body,
    grid=(B // BB,),
    in_specs=[
        pl.BlockSpec((BB, TAG), lambda i: (i, 0)),
        pl.BlockSpec((BB, TAG), lambda i: (i, 0)),
        pl.BlockSpec((BB, TAG), lambda i: (i, 0)),
        _full((TAG, D)),
        _full((D, TAG)),
        _full((D, TAG)),
        _full((D, TAG)),
        _full((1, TAG)),
        _full((TAG, D)),
        _full((1, D)),
        _full((D, 1)),
        _full((1, 1)),
    ],
    out_specs=pl.BlockSpec((BB, 1), lambda i: (i, 0)),
    out_shape=jax.ShapeDtypeStruct((B, 1), jnp.float32),
)


def kernel(user_idx, artist_idx, tag_features, user_emb, artist_emb,
           W_tag, W1, b1, W2, b2, W3, b3):
    ui = user_idx.astype(jnp.int32)
    ai = artist_idx.astype(jnp.int32)
    u2 = _gather_user(ui, user_emb.T)
    a2 = _gather_artist(ai, artist_emb.T)
    out = _mlp(u2, a2, tag_features,
               W_tag.T,
               W1[:, :D].T, W1[:, D:2 * D].T, W1[:, 2 * D:].T,
               b1.reshape(1, -1),
               W2.T, b2.reshape(1, -1),
               W3.T, b3.reshape(1, 1))
    return out.reshape(B)


# speedup vs baseline: 1.1938x; 1.1938x over previous
import functools

import jax
import jax.numpy as jnp
from jax import lax
from jax.experimental import pallas as pl
from jax.experimental.pallas import tpu as pltpu
from jax.experimental.pallas import tpu_sc as plsc

B = 16384
D = 64
TAG = 128
NW = 32
BOUT = B + NW
LANES = 16


def _splat(x, dtype=jnp.int32):
    return jnp.full((LANES,), x, dtype)


def _make_stream_gather(V, WV, WVS):
    assert WV == 1 << WVS
    n_full = V // WV
    tail_w = V - n_full * WV
    tail_off = n_full * WV

    def body(idx_hbm, tableT_hbm, out_hbm,
             ibuf, hbuf, winbuf, tailbuf, curl,
             staging, posline, semw):
        wid = lax.axis_index("s") * 2 + lax.axis_index("c")
        kmax = (n_full - 1 - wid) // NW + 1
        iota = lax.iota(jnp.int32, LANES)

        def zrow(i, c):
            for j in range(4):
                plsc.store_scatter(staging,
                                   [_splat(i), D + 16 * j + iota],
                                   jnp.zeros((LANES,), jnp.float32))
            return c
        lax.fori_loop(0, 128, zrow, 0)

        for j in range(8):
            plsc.store_scatter(posline, [_splat(0), 16 * j + iota],
                               _splat(B + wid))

        IC = 2048

        def chunk(q, off):
            pltpu.sync_copy(idx_hbm.at[pl.ds(q * IC, IC)], ibuf)

            def scan(g, off):
                v = ibuf[pl.ds(g * LANES, LANES)]
                pos = q * IC + g * LANES + iota
                m = ((v >> WVS) & (NW - 1)) == wid
                packed = ((v >> (WVS + 5)) << 23) | ((v & (WV - 1)) << 14) | pos
                plsc.store_compressed(hbuf.at[pl.ds(off, LANES)], packed,
                                      mask=m)
                return off + plsc.all_reduce_population_count(m)[0]

            return lax.fori_loop(0, IC // LANES, scan, off)

        n = lax.fori_loop(0, B // IC, chunk, 0)
        hbuf[pl.ds(n, LANES)] = _splat(jnp.int32(2147400000))

        nvec = (n + LANES - 1) // LANES

        def process_hits(k_id, slot_idx, src_ref, f):
            def hit_vec(t, f):
                hv = hbuf[pl.ds(t * LANES, LANES)]
                m = (hv >> 23) == k_id
                plsc.store_compressed(curl.at[pl.ds(0, LANES)], hv, mask=m)
                c = plsc.all_reduce_population_count(m)[0]

                def item(j, f):
                    w = curl[pl.ds(j, LANES)][0]
                    l = (w >> 14) & 511
                    p = w & (B - 1)
                    fm = lax.rem(f, 128)
                    for i in range(4):
                        dvec = 16 * i + iota
                        if src_ref is winbuf:
                            vals = plsc.load_gather(
                                winbuf, [_splat(slot_idx), dvec, _splat(l)])
                        else:
                            vals = plsc.load_gather(src_ref, [dvec, _splat(l)])
                        plsc.store_scatter(staging, [_splat(fm), dvec], vals)
                    plsc.store_scatter(posline, [_splat(0), _splat(fm)],
                                       _splat(p), mask=iota == 0)
                    f = f + 1

                    @pl.when(lax.rem(f, 128) == 0)
                    def _flush():
                        pltpu.sync_copy(staging, out_hbm.at[posline.at[0]])
                    return f

                return lax.fori_loop(0, c, item, f)

            return lax.fori_loop(0, nvec, hit_vec, f)

        def win_src(k):
            off = pl.multiple_of((wid + k * NW) * WV, WV)
            return tableT_hbm.at[:, pl.ds(off, WV)]

        pltpu.async_copy(win_src(0), winbuf.at[0], semw)

        def wloop(k, f):
            @pl.when(k + 1 < kmax)
            def _():
                pltpu.async_copy(win_src(k + 1), winbuf.at[lax.rem(k + 1, 2)],
                                 semw)
            pltpu.make_async_copy(win_src(k), winbuf.at[lax.rem(k, 2)],
                                  semw).wait()
            return process_hits(k, lax.rem(k, 2), winbuf, f)

        f = lax.fori_loop(0, kmax, wloop, 0)

        if tail_w:
            pltpu.sync_copy(tableT_hbm.at[:, pl.ds(tail_off, tail_w)], tailbuf)
            f = process_hits(n_full // NW, 0, tailbuf, f)

        @pl.when(lax.rem(f, 128) != 0)
        def _():
            pltpu.sync_copy(staging, out_hbm.at[posline.at[0]])

    return functools.partial(
        pl.kernel,
        mesh=plsc.VectorSubcoreMesh(core_axis_name="c", subcore_axis_name="s"),
        out_type=jax.ShapeDtypeStruct((BOUT, TAG), jnp.float32),
        scratch_types=[
            pltpu.VMEM((2048,), jnp.int32),
            pltpu.VMEM((B + LANES,), jnp.int32),
            pltpu.VMEM((2, D, WV), jnp.float32),
            pltpu.VMEM((D, max(tail_w, 8)), jnp.float32),
            pltpu.VMEM((2 * LANES,), jnp.int32),
            pltpu.VMEM((128, TAG), jnp.float32),
            pltpu.VMEM((1, TAG), jnp.int32),
            pltpu.SemaphoreType.DMA,
        ],
        compiler_params=pltpu.CompilerParams(use_tc_tiling_on_sc=True, needs_layout_passes=False),
    )(body)


_gather_user = _make_stream_gather(1000000, 512, 9)
_gather_artist = _make_stream_gather(100000, 256, 8)


BB = 2048


def _mlp_body(u2_ref, a2_ref, t_ref, wtag_ref, w1u_ref, w1a_ref, w1t_ref,
              b1_ref, w2_ref, b2_ref, w3_ref, b3_ref, out_ref):
    f32 = jnp.float32
    u = u2_ref[:, :D]
    a = a2_ref[:, :D]
    t = jnp.maximum(
        jnp.dot(t_ref[...], wtag_ref[...], preferred_element_type=f32), 0.0)
    h = jnp.dot(u, w1u_ref[...], preferred_element_type=f32)
    h = h + jnp.dot(a, w1a_ref[...], preferred_element_type=f32)
    h = h + jnp.dot(t, w1t_ref[...], preferred_element_type=f32)
    h = jnp.maximum(h + b1_ref[...], 0.0)
    h = jnp.maximum(
        jnp.dot(h, w2_ref[...], preferred_element_type=f32) + b2_ref[...], 0.0)
    logit = jnp.dot(h, w3_ref[...], preferred_element_type=f32) + b3_ref[...]
    out_ref[...] = jax.nn.sigmoid(logit)


def _full(shape):
    return pl.BlockSpec(shape, lambda i: (0, 0))


_mlp = pl.pallas_call(
    _mlp_body,
    grid=(B // BB,),
    in_specs=[
        pl.BlockSpec((BB, TAG), lambda i: (i, 0)),
        pl.BlockSpec((BB, TAG), lambda i: (i, 0)),
        pl.BlockSpec((BB, TAG), lambda i: (i, 0)),
        _full((TAG, D)),
        _full((D, TAG)),
        _full((D, TAG)),
        _full((D, TAG)),
        _full((1, TAG)),
        _full((TAG, D)),
        _full((1, D)),
        _full((D, 1)),
        _full((1, 1)),
    ],
    out_specs=pl.BlockSpec((BB, 1), lambda i: (i, 0)),
    out_shape=jax.ShapeDtypeStruct((B, 1), jnp.float32),
)


def kernel(user_idx, artist_idx, tag_features, user_emb, artist_emb,
           W_tag, W1, b1, W2, b2, W3, b3):
    ui = user_idx.astype(jnp.int32)
    ai = artist_idx.astype(jnp.int32)
    u2 = _gather_user(ui, user_emb.T)
    a2 = _gather_artist(ai, artist_emb.T)
    out = _mlp(u2, a2, tag_features,
               W_tag.T,
               W1[:, :D].T, W1[:, D:2 * D].T, W1[:, 2 * D:].T,
               b1.reshape(1, -1),
               W2.T, b2.reshape(1, -1),
               W3.T, b3.reshape(1, 1))
    return out.reshape(B)


# speedup vs baseline: 1.2137x; 1.0166x over previous
import functools

import jax
import jax.numpy as jnp
from jax import lax
from jax.experimental import pallas as pl
from jax.experimental.pallas import tpu as pltpu
from jax.experimental.pallas import tpu_sc as plsc

B = 16384
D = 64
TAG = 128
NW = 32
BOUT = B + NW
LANES = 16


def _splat(x, dtype=jnp.int32):
    return jnp.full((LANES,), x, dtype)


def _make_stream_gather(V, WV, WVS):
    assert WV == 1 << WVS
    n_full = V // WV
    tail_w = V - n_full * WV
    tail_off = n_full * WV

    def body(idx_hbm, tableT_hbm, out_hbm,
             ibuf, hbuf, winbuf, tailbuf, curl,
             staging, posline, semw):
        wid = lax.axis_index("s") * 2 + lax.axis_index("c")
        kmax = (n_full - 1 - wid) // NW + 1
        iota = lax.iota(jnp.int32, LANES)

        def zrow(i, c):
            for j in range(4):
                plsc.store_scatter(staging,
                                   [_splat(i), D + 16 * j + iota],
                                   jnp.zeros((LANES,), jnp.float32))
            return c
        lax.fori_loop(0, 128, zrow, 0)

        for j in range(8):
            plsc.store_scatter(posline, [_splat(0), 16 * j + iota],
                               _splat(B + wid))

        IC = 2048

        def chunk(q, off):
            pltpu.sync_copy(idx_hbm.at[pl.ds(q * IC, IC)], ibuf)

            def scan(g, off):
                v = ibuf[pl.ds(g * LANES, LANES)]
                pos = q * IC + g * LANES + iota
                m = ((v >> WVS) & (NW - 1)) == wid
                tailbit = (v >= tail_off).astype(jnp.int32)
                packed = ((tailbit << 30) | ((v >> (WVS + 5)) << 23)
                          | ((v & (WV - 1)) << 14) | pos)
                plsc.store_compressed(hbuf.at[pl.ds(off, LANES)], packed,
                                      mask=m)
                return off + plsc.all_reduce_population_count(m)[0]

            return lax.fori_loop(0, IC // LANES, scan, off)

        n = lax.fori_loop(0, B // IC, chunk, 0)
        hbuf[pl.ds(n, LANES)] = _splat(jnp.int32(2147400000))

        nvec = (n + LANES - 1) // LANES

        def process_hits(k_id, slot_idx, src_ref, f):
            def hit_vec(t, f):
                hv = hbuf[pl.ds(t * LANES, LANES)]
                m = (hv >> 23) == k_id
                plsc.store_compressed(curl.at[pl.ds(0, LANES)], hv, mask=m)
                c = plsc.all_reduce_population_count(m)[0]

                def item(j, f):
                    w = curl[pl.ds(j, LANES)][0]
                    l = (w >> 14) & 511
                    p = w & (B - 1)
                    fm = lax.rem(f, 128)
                    for i in range(4):
                        dvec = 16 * i + iota
                        if src_ref is winbuf:
                            vals = plsc.load_gather(
                                winbuf, [_splat(slot_idx), dvec, _splat(l)])
                        else:
                            vals = plsc.load_gather(src_ref, [dvec, _splat(l)])
                        plsc.store_scatter(staging, [_splat(fm), dvec], vals)
                    plsc.store_scatter(posline, [_splat(0), _splat(fm)],
                                       _splat(p), mask=iota == 0)
                    f = f + 1

                    @pl.when(lax.rem(f, 128) == 0)
                    def _flush():
                        pltpu.sync_copy(staging, out_hbm.at[posline.at[0]])
                    return f

                return lax.fori_loop(0, c, item, f)

            return lax.fori_loop(0, nvec, hit_vec, f)

        def win_src(k):
            off = pl.multiple_of((wid + k * NW) * WV, WV)
            return tableT_hbm.at[:, pl.ds(off, WV)]

        pltpu.async_copy(win_src(0), winbuf.at[0], semw)

        def wloop(k, f):
            @pl.when(k + 1 < kmax)
            def _():
                pltpu.async_copy(win_src(k + 1), winbuf.at[lax.rem(k + 1, 2)],
                                 semw)
            pltpu.make_async_copy(win_src(k), winbuf.at[lax.rem(k, 2)],
                                  semw).wait()
            return process_hits(k, lax.rem(k, 2), winbuf, f)

        f = lax.fori_loop(0, kmax, wloop, 0)

        if tail_w:
            pltpu.sync_copy(tableT_hbm.at[:, pl.ds(tail_off, tail_w)], tailbuf)
            f = process_hits(n_full // NW + 128, 0, tailbuf, f)

        @pl.when(lax.rem(f, 128) != 0)
        def _():
            pltpu.sync_copy(staging, out_hbm.at[posline.at[0]])

    return functools.partial(
        pl.kernel,
        mesh=plsc.VectorSubcoreMesh(core_axis_name="c", subcore_axis_name="s"),
        out_type=jax.ShapeDtypeStruct((BOUT, TAG), jnp.float32),
        scratch_types=[
            pltpu.VMEM((2048,), jnp.int32),
            pltpu.VMEM((B + LANES,), jnp.int32),
            pltpu.VMEM((2, D, WV), jnp.float32),
            pltpu.VMEM((D, max(tail_w, 8)), jnp.float32),
            pltpu.VMEM((2 * LANES,), jnp.int32),
            pltpu.VMEM((128, TAG), jnp.float32),
            pltpu.VMEM((1, TAG), jnp.int32),
            pltpu.SemaphoreType.DMA,
        ],
        compiler_params=pltpu.CompilerParams(use_tc_tiling_on_sc=True, needs_layout_passes=False),
    )(body)


_gather_user = _make_stream_gather(1000000, 512, 9)
_gather_artist = _make_stream_gather(100000, 256, 8)


BB = 2048


def _mlp_body(u2_ref, a2_ref, t_ref, wtag_ref, w1u_ref, w1a_ref, w1t_ref,
              b1_ref, w2_ref, b2_ref, w3_ref, b3_ref, out_ref):
    f32 = jnp.float32
    u = u2_ref[:, :D]
    a = a2_ref[:, :D]
    t = jnp.maximum(
        jnp.dot(t_ref[...], wtag_ref[...], preferred_element_type=f32), 0.0)
    h = jnp.dot(u, w1u_ref[...], preferred_element_type=f32)
    h = h + jnp.dot(a, w1a_ref[...], preferred_element_type=f32)
    h = h + jnp.dot(t, w1t_ref[...], preferred_element_type=f32)
    h = jnp.maximum(h + b1_ref[...], 0.0)
    h = jnp.maximum(
        jnp.dot(h, w2_ref[...], preferred_element_type=f32) + b2_ref[...], 0.0)
    logit = jnp.dot(h, w3_ref[...], preferred_element_type=f32) + b3_ref[...]
    out_ref[...] = jax.nn.sigmoid(logit)


def _full(shape):
    return pl.BlockSpec(shape, lambda i: (0, 0))


_mlp = pl.pallas_call(
    _mlp_body,
    grid=(B // BB,),
    in_specs=[
        pl.BlockSpec((BB, TAG), lambda i: (i, 0)),
        pl.BlockSpec((BB, TAG), lambda i: (i, 0)),
        pl.BlockSpec((BB, TAG), lambda i: (i, 0)),
        _full((TAG, D)),
        _full((D, TAG)),
        _full((D, TAG)),
        _full((D, TAG)),
        _full((1, TAG)),
        _full((TAG, D)),
        _full((1, D)),
        _full((D, 1)),
        _full((1, 1)),
    ],
    out_specs=pl.BlockSpec((BB, 1), lambda i: (i, 0)),
    out_shape=jax.ShapeDtypeStruct((B, 1), jnp.float32),
)


def kernel(user_idx, artist_idx, tag_features, user_emb, artist_emb,
           W_tag, W1, b1, W2, b2, W3, b3):
    ui = user_idx.astype(jnp.int32)
    ai = artist_idx.astype(jnp.int32)
    u2 = _gather_user(ui, user_emb.T)
    a2 = _gather_artist(ai, artist_emb.T)
    out = _mlp(u2, a2, tag_features,
               W_tag.T,
               W1[:, :D].T, W1[:, D:2 * D].T, W1[:, 2 * D:].T,
               b1.reshape(1, -1),
               W2.T, b2.reshape(1, -1),
               W3.T, b3.reshape(1, 1))
    return out.reshape(B)
